# baseline (speedup 1.0000x reference)
import functools

import jax
import jax.numpy as jnp
from jax import lax
from jax.experimental import pallas as pl
from jax.experimental.pallas import tpu as pltpu
from jax.experimental.pallas import tpu_sc as plsc

_SPATIAL = (21, 200, 176)
_K = 3
_S = 2
_P = 1
_CIN = 16
_COUT = 32
_BATCH = 2
_NK = _K ** 3

_N = 200000
_NP = 204800
_B = _NP * _NK
_NC, _NS = 2, 16
_NW = _NC * _NS
_PW = _B // _NW
_SUB = 128
_GRP = 10
_CB = _SUB * _GRP
_CHUNKS = _PW // _CB
_DUMMY = 4096


def _out_dim(d):
    return (d + 2 * _P - (_K - 1) - 1) // _S + 1


_DO, _HO, _WO = _out_dim(_SPATIAL[0]), _out_dim(_SPATIAL[1]), _out_dim(_SPATIAL[2])
_TOTAL = _BATCH * _DO * _HO * _WO
_R = _TOTAL + _DUMMY

_GD, _GH, _GW = _SPATIAL[0] + 2, _SPATIAL[1] + 2, _SPATIAL[2] + 2


def _pairs(coors):
    offs = jnp.array([(a, b, c) for a in range(_K) for b in range(_K) for c in range(_K)],
                     dtype=jnp.int32)
    pos = coors[:, 1:4].astype(jnp.int32)
    b = coors[:, 0].astype(jnp.int32)
    num = pos[:, None, :] + _P - offs[None, :, :]
    q = num // _S
    r = num - q * _S
    odims = jnp.array([_DO, _HO, _WO], dtype=jnp.int32)
    valid = (r == 0).all(-1) & (q >= 0).all(-1) & (q < odims[None, None, :]).all(-1)
    lin = b[:, None] * (_DO * _HO * _WO) + q[..., 0] * (_HO * _WO) + q[..., 1] * _WO + q[..., 2]
    n = coors.shape[0]
    spread = (jnp.arange(n * _NK, dtype=jnp.int32) % _DUMMY).reshape(n, _NK)
    lin_safe = jnp.where(valid, lin, _TOTAL + spread)
    return lin_safe


def _mmacc_body(x_ref, w_ref, out_ref):
    @pl.when(pl.program_id(1) == 0)
    def _init():
        out_ref[...] = jnp.zeros_like(out_ref)

    out_ref[...] = out_ref[...] + jax.lax.dot(
        x_ref[0], w_ref[0], preferred_element_type=jnp.float32)


def _mmacc(x, w, bn):
    nk, n, cin = x.shape
    cout = w.shape[2]
    return pl.pallas_call(
        _mmacc_body,
        grid=(n // bn, nk),
        in_specs=[
            pl.BlockSpec((1, bn, cin), lambda i, k: (k, i, 0)),
            pl.BlockSpec((1, cin, cout), lambda i, k: (k, 0, 0)),
        ],
        out_specs=pl.BlockSpec((bn, cout), lambda i, k: (i, 0)),
        out_shape=jax.ShapeDtypeStruct((n, cout), jnp.float32),
    )(x, w)


def _mm_body(x_ref, w_ref, out_ref):
    out_ref[...] = jax.lax.dot(x_ref[...], w_ref[...],
                               preferred_element_type=jnp.float32)


def _mm(x, w, bn):
    n, cin = x.shape
    cout = w.shape[1]
    return pl.pallas_call(
        _mm_body,
        grid=(n // bn,),
        in_specs=[
            pl.BlockSpec((bn, cin), lambda i: (i, 0)),
            pl.BlockSpec((cin, cout), lambda i: (0, 0)),
        ],
        out_specs=pl.BlockSpec((bn, cout), lambda i: (i, 0)),
        out_shape=jax.ShapeDtypeStruct((n, cout), jnp.float32),
    )(x, w)


def _gather_body(idx_hbm, table_hbm, out_hbm, idx_v, rows_v, sem):
    wid = lax.axis_index("s") * _NC + lax.axis_index("c")
    out_row0 = wid * _PW

    def step(j, carry):
        pltpu.sync_copy(idx_hbm.at[wid * _CHUNKS + j], idx_v)
        copies = []
        for g in range(_GRP):
            copies.append(pltpu.async_copy(
                table_hbm.at[idx_v.at[g]],
                rows_v.at[pl.ds(g * _SUB, _SUB)], sem))
        for c in copies:
            c.wait()
        pltpu.sync_copy(rows_v, out_hbm.at[pl.ds(out_row0 + j * _CB, _CB)])
        return carry

    lax.fori_loop(0, _CHUNKS, step, 0)


_gather_sc = functools.partial(
    pl.kernel,
    out_type=jax.ShapeDtypeStruct((_B, _COUT), jnp.float32),
    mesh=plsc.VectorSubcoreMesh(core_axis_name="c", subcore_axis_name="s"),
    scratch_types=[
        pltpu.VMEM((_GRP, _SUB), jnp.int32),
        pltpu.VMEM((_CB, _COUT), jnp.float32),
        pltpu.SemaphoreType.DMA,
    ],
    compiler_params=pltpu.CompilerParams(use_tc_tiling_on_sc=False),
)(_gather_body)


@jax.jit
def kernel(features, coors, W_conv, W_inv):
    coors32 = coors.astype(jnp.int32)
    b = coors32[:, 0]
    z = coors32[:, 1]
    y = coors32[:, 2]
    x = coors32[:, 3]

    lin_pad = ((b * _GD + (z + 1)) * _GH + (y + 1)) * _GW + (x + 1)
    Fd = jnp.zeros((_BATCH * _GD * _GH * _GW, _CIN), jnp.float32)
    Fd = Fd.at[lin_pad].add(features)
    Fd5 = Fd.reshape(_BATCH, _GD, _GH, _GW, _CIN)

    slices = []
    for a in range(_K):
        for bb in range(_K):
            for c in range(_K):
                slices.append(lax.slice(
                    Fd5,
                    (0, a, bb, c, 0),
                    (_BATCH, a + _S * _DO - 1, bb + _S * _HO - 1, c + _S * _WO - 1, _CIN),
                    (1, _S, _S, _S, 1)))
    X = jnp.stack(slices, axis=0).reshape(_NK, _TOTAL, _CIN)

    out_tc = _mmacc(X, W_conv, 800)
    table = jnp.concatenate(
        [out_tc, jnp.zeros((_DUMMY, _COUT), jnp.float32)], axis=0)

    pad_coors = jnp.tile(jnp.array([[0, 25, 0, 0]], jnp.int32), (_NP - _N, 1))
    coors_p = jnp.concatenate([coors32, pad_coors], axis=0)
    lin_safe = _pairs(coors_p)

    G = _gather_sc(lin_safe.reshape(_B // _CB, _GRP, _SUB), table)

    Wbig = jnp.flip(W_inv, 0).reshape(_NK * _COUT, _CIN)
    res = _mm(G.reshape(_NP, _NK * _COUT), Wbig, 1024)
    return res[:_N]


# speedup vs baseline: 2.8861x; 2.8861x over previous
import functools

import jax
import jax.numpy as jnp
from jax import lax
from jax.experimental import pallas as pl
from jax.experimental.pallas import tpu as pltpu
from jax.experimental.pallas import tpu_sc as plsc

_SPATIAL = (21, 200, 176)
_K = 3
_S = 2
_P = 1
_CIN = 16
_COUT = 32
_BATCH = 2
_NK = _K ** 3

_N = 200000
_NP = 204800
_B = _NP * _NK
_NC, _NS = 2, 16
_NW = _NC * _NS
_PW = _B // _NW
_SUB = 128
_GRP = 10
_CB = _SUB * _GRP
_CHUNKS = _PW // _CB
_DUMMY = 4096


def _out_dim(d):
    return (d + 2 * _P - (_K - 1) - 1) // _S + 1


_DO, _HO, _WO = _out_dim(_SPATIAL[0]), _out_dim(_SPATIAL[1]), _out_dim(_SPATIAL[2])
_TOTAL = _BATCH * _DO * _HO * _WO
_R = _TOTAL + _DUMMY

_MZ, _MY, _MX = 12, 101, 89


def _pairs(coors):
    offs = jnp.array([(a, b, c) for a in range(_K) for b in range(_K) for c in range(_K)],
                     dtype=jnp.int32)
    pos = coors[:, 1:4].astype(jnp.int32)
    b = coors[:, 0].astype(jnp.int32)
    num = pos[:, None, :] + _P - offs[None, :, :]
    q = num // _S
    r = num - q * _S
    odims = jnp.array([_DO, _HO, _WO], dtype=jnp.int32)
    valid = (r == 0).all(-1) & (q >= 0).all(-1) & (q < odims[None, None, :]).all(-1)
    lin = b[:, None] * (_DO * _HO * _WO) + q[..., 0] * (_HO * _WO) + q[..., 1] * _WO + q[..., 2]
    n = coors.shape[0]
    spread = (jnp.arange(n * _NK, dtype=jnp.int32) % _DUMMY).reshape(n, _NK)
    lin_safe = jnp.where(valid, lin, _TOTAL + spread)
    return lin_safe


def _mmk_body(x_ref, w_ref, out_ref):
    acc = jax.lax.dot(x_ref[0], w_ref[0], preferred_element_type=jnp.float32)
    for k in range(1, _NK):
        acc = acc + jax.lax.dot(x_ref[k], w_ref[k],
                                preferred_element_type=jnp.float32)
    out_ref[...] = acc


def _mmk(x, w, bn):
    nk, n, cin = x.shape
    cout = w.shape[2]
    return pl.pallas_call(
        _mmk_body,
        grid=(n // bn,),
        in_specs=[
            pl.BlockSpec((nk, bn, cin), lambda i: (0, i, 0)),
            pl.BlockSpec((nk, cin, cout), lambda i: (0, 0, 0)),
        ],
        out_specs=pl.BlockSpec((bn, cout), lambda i: (i, 0)),
        out_shape=jax.ShapeDtypeStruct((n, cout), jnp.float32),
    )(x, w)


def _mm_body(x_ref, w_ref, out_ref):
    out_ref[...] = jax.lax.dot(x_ref[...], w_ref[...],
                               preferred_element_type=jnp.float32)


def _mm(x, w, bn):
    n, cin = x.shape
    cout = w.shape[1]
    return pl.pallas_call(
        _mm_body,
        grid=(n // bn,),
        in_specs=[
            pl.BlockSpec((bn, cin), lambda i: (i, 0)),
            pl.BlockSpec((cin, cout), lambda i: (0, 0)),
        ],
        out_specs=pl.BlockSpec((bn, cout), lambda i: (i, 0)),
        out_shape=jax.ShapeDtypeStruct((n, cout), jnp.float32),
    )(x, w)


def _gather_body(idx_hbm, table_hbm, out_hbm, idx_v, rows_v, sem):
    wid = lax.axis_index("s") * _NC + lax.axis_index("c")
    out_row0 = wid * _PW

    def step(j, carry):
        pltpu.sync_copy(idx_hbm.at[wid * _CHUNKS + j], idx_v)
        copies = []
        for g in range(_GRP):
            copies.append(pltpu.async_copy(
                table_hbm.at[idx_v.at[g]],
                rows_v.at[pl.ds(g * _SUB, _SUB)], sem))
        for c in copies:
            c.wait()
        pltpu.sync_copy(rows_v, out_hbm.at[pl.ds(out_row0 + j * _CB, _CB)])
        return carry

    lax.fori_loop(0, _CHUNKS, step, 0)


_gather_sc = functools.partial(
    pl.kernel,
    out_type=jax.ShapeDtypeStruct((_B, _COUT), jnp.float32),
    mesh=plsc.VectorSubcoreMesh(core_axis_name="c", subcore_axis_name="s"),
    scratch_types=[
        pltpu.VMEM((_GRP, _SUB), jnp.int32),
        pltpu.VMEM((_CB, _COUT), jnp.float32),
        pltpu.SemaphoreType.DMA,
    ],
    compiler_params=pltpu.CompilerParams(use_tc_tiling_on_sc=False),
)(_gather_body)


@jax.jit
def kernel(features, coors, W_conv, W_inv):
    coors32 = coors.astype(jnp.int32)
    b = coors32[:, 0]
    z = coors32[:, 1]
    y = coors32[:, 2]
    x = coors32[:, 3]

    gz, gy, gx = z + 1, y + 1, x + 1
    pc = (gz & 1) * 4 + (gy & 1) * 2 + (gx & 1)
    lin_pad = ((((pc * _BATCH + b) * _MZ + (gz >> 1)) * _MY + (gy >> 1)) * _MX
               + (gx >> 1))
    Fd = jnp.zeros((8 * _BATCH * _MZ * _MY * _MX, _CIN), jnp.float32)
    Fd = Fd.at[lin_pad].add(features)
    Fd6 = Fd.reshape(8, _BATCH, _MZ, _MY, _MX, _CIN)

    slices = []
    for a in range(_K):
        for bb in range(_K):
            for c in range(_K):
                p = (a & 1) * 4 + (bb & 1) * 2 + (c & 1)
                slices.append(lax.slice(
                    Fd6,
                    (p, 0, a >> 1, bb >> 1, c >> 1, 0),
                    (p + 1, _BATCH, (a >> 1) + _DO, (bb >> 1) + _HO,
                     (c >> 1) + _WO, _CIN)))
    X = jnp.concatenate(slices, axis=0).reshape(_NK, _TOTAL, _CIN)

    out_tc = _mmk(X, W_conv, 800)
    table = jnp.concatenate(
        [out_tc, jnp.zeros((_DUMMY, _COUT), jnp.float32)], axis=0)

    pad_coors = jnp.tile(jnp.array([[0, 25, 0, 0]], jnp.int32), (_NP - _N, 1))
    coors_p = jnp.concatenate([coors32, pad_coors], axis=0)
    lin_safe = _pairs(coors_p)

    G = _gather_sc(lin_safe.reshape(_B // _CB, _GRP, _SUB), table)

    Wbig = jnp.flip(W_inv, 0).reshape(_NK * _COUT, _CIN)
    res = _mm(G.reshape(_NP, _NK * _COUT), Wbig, 1024)
    return res[:_N]
